# baseline (speedup 1.0000x reference)
import functools

import jax
import jax.numpy as jnp
from jax import lax
from jax.experimental import pallas as pl
from jax.experimental.pallas import tpu as pltpu
from jax.experimental.pallas import tpu_sc as plsc

N_NODES = 10000
N_EDGES = 320000
D = 128
BN_EPS = 1e-5

NC = 2
NS = 16
NW = NC * NS
CHUNK = 80
CPW = N_EDGES // (NW * CHUNK)
ROWS_PER_TILE = 640
NPAD = NS * ROWS_PER_TILE


def _sc_aggregate(with_deg):
    mesh = plsc.VectorSubcoreMesh(
        core_axis_name="c", subcore_axis_name="s",
        num_cores=NC, num_subcores=NS)

    out_type = [jax.ShapeDtypeStruct((NC, NPAD, D), jnp.float32)]
    scratch = [
        pltpu.VMEM_SHARED((NPAD, D), jnp.float32),
        pltpu.VMEM((CPW * CHUNK,), jnp.int32),
        pltpu.VMEM((CHUNK, D), jnp.float32),
        pltpu.VMEM((CHUNK, D), jnp.float32),
        pltpu.VMEM((CHUNK, D), jnp.float32),
        pltpu.VMEM((CHUNK,), jnp.int32),
        pltpu.VMEM((CHUNK,), jnp.int32),
        pltpu.VMEM((CHUNK,), jnp.int32),
        pltpu.SemaphoreType.DMA,
        pltpu.SemaphoreType.DMA,
        pltpu.SemaphoreType.DMA,
        pltpu.SemaphoreType.DMA,
        pltpu.SemaphoreType.DMA,
        pltpu.SemaphoreType.DMA,
        pltpu.SemaphoreType.DMA,
        pltpu.SemaphoreType.DMA,
        pltpu.SemaphoreType.DMA,
    ]
    if with_deg:
        out_type.append(jax.ShapeDtypeStruct((NC, NPAD), jnp.float32))
        scratch += [
            pltpu.VMEM_SHARED((NPAD,), jnp.float32),
            pltpu.VMEM((CHUNK,), jnp.float32),
            pltpu.VMEM((ROWS_PER_TILE,), jnp.float32),
            pltpu.SemaphoreType.DMA,
            pltpu.SemaphoreType.DMA,
            pltpu.SemaphoreType.DMA,
        ]

    def body(xw_hbm, ei_hbm, parts_hbm, *rest):
        if with_deg:
            (degp_hbm, acc, srcv, r0, r1, r2, d0, d1, d2,
             g0, g1, g2, s0, s1, s2, i0, i1, i2,
             dacc, ones, dz, e0, e1, e2) = rest
            semd = (e0, e1, e2)
        else:
            (acc, srcv, r0, r1, r2, d0, d1, d2,
             g0, g1, g2, s0, s1, s2, i0, i1, i2) = rest
        rows = (r0, r1, r2)
        dsti = (d0, d1, d2)
        semg = (g0, g1, g2)
        sems = (s0, s1, s2)
        semi = (i0, i1, i2)

        c = lax.axis_index("c")
        s = lax.axis_index("s")
        wid = s * NC + c
        row0 = s * ROWS_PER_TILE
        ebase = wid * CPW * CHUNK

        pltpu.async_copy(ei_hbm.at[0, pl.ds(ebase, CPW * CHUNK)], srcv,
                         semi[2])

        z16 = jnp.zeros((16,), jnp.float32)

        def zrow(i, carry):
            for j in range(D // 16):
                r0[i, pl.ds(j * 16, 16)] = z16
            return carry
        lax.fori_loop(0, CHUNK, zrow, 0)
        nz = ROWS_PER_TILE // CHUNK
        for k in range(nz):
            pltpu.async_copy(r0, acc.at[pl.ds(row0 + k * CHUNK, CHUNK)],
                             sems[k % 3])

        if with_deg:
            o16 = jnp.ones((16,), jnp.float32)

            def zdeg(i, carry):
                dz[pl.ds(i * 16, 16)] = z16
                return carry
            lax.fori_loop(0, ROWS_PER_TILE // 16, zdeg, 0)
            pltpu.async_copy(dz, dacc.at[pl.ds(row0, ROWS_PER_TILE)],
                             semd[0])

            def fones(i, carry):
                ones[pl.ds(i * 16, 16)] = o16
                return carry
            lax.fori_loop(0, CHUNK // 16, fones, 0)
            pltpu.make_async_copy(dz, dacc.at[pl.ds(row0, ROWS_PER_TILE)],
                                  semd[0]).wait()

        for k in range(nz):
            pltpu.make_async_copy(r0, acc.at[pl.ds(row0 + k * CHUNK, CHUNK)],
                                  sems[k % 3]).wait()
        pltpu.make_async_copy(ei_hbm.at[0, pl.ds(ebase, CPW * CHUNK)], srcv,
                              semi[2]).wait()

        plsc.subcore_barrier()

        def fetch(j, b):
            pltpu.async_copy(ei_hbm.at[1, pl.ds(ebase + j * CHUNK, CHUNK)],
                             dsti[b], semi[b])
            pltpu.async_copy(xw_hbm.at[srcv.at[pl.ds(j * CHUNK, CHUNK)]],
                             rows[b], semg[b])

        def wait_fetch(j, b):
            pltpu.make_async_copy(
                ei_hbm.at[1, pl.ds(ebase + j * CHUNK, CHUNK)],
                dsti[b], semi[b]).wait()
            pltpu.make_async_copy(xw_hbm.at[srcv.at[pl.ds(j * CHUNK, CHUNK)]],
                                  rows[b], semg[b]).wait()

        def scatter(b):
            pltpu.async_copy(rows[b], acc.at[dsti[b]], sems[b], add=True)
            if with_deg:
                pltpu.async_copy(ones, dacc.at[dsti[b]], semd[b], add=True)

        def wait_scatter(b):
            pltpu.make_async_copy(rows[b], acc.at[dsti[b]], sems[b]).wait()
            if with_deg:
                pltpu.make_async_copy(ones, dacc.at[dsti[b]],
                                      semd[b]).wait()

        fetch(0, 0)
        fetch(1, 1)

        def step(i, carry):
            j = 3 * i
            wait_fetch(j, 0)
            scatter(0)

            @pl.when(i > 0)
            def _():
                wait_scatter(2)
            fetch(j + 2, 2)

            wait_fetch(j + 1, 1)
            scatter(1)
            wait_scatter(0)
            fetch(j + 3, 0)

            wait_fetch(j + 2, 2)
            scatter(2)
            wait_scatter(1)
            fetch(j + 4, 1)
            return carry
        lax.fori_loop(0, (CPW - 2) // 3, step, 0)

        wait_fetch(CPW - 2, 0)
        scatter(0)
        wait_fetch(CPW - 1, 1)
        scatter(1)
        wait_scatter(2)
        wait_scatter(0)
        wait_scatter(1)

        plsc.subcore_barrier()

        pltpu.async_copy(acc.at[pl.ds(row0, ROWS_PER_TILE)],
                         parts_hbm.at[c, pl.ds(row0, ROWS_PER_TILE)],
                         sems[0])
        if with_deg:
            pltpu.async_copy(dacc.at[pl.ds(row0, ROWS_PER_TILE)],
                             degp_hbm.at[c, pl.ds(row0, ROWS_PER_TILE)],
                             semd[0])
            pltpu.make_async_copy(
                dacc.at[pl.ds(row0, ROWS_PER_TILE)],
                degp_hbm.at[c, pl.ds(row0, ROWS_PER_TILE)], semd[0]).wait()
        pltpu.make_async_copy(
            acc.at[pl.ds(row0, ROWS_PER_TILE)],
            parts_hbm.at[c, pl.ds(row0, ROWS_PER_TILE)], sems[0]).wait()

    return pl.kernel(
        body, out_type=out_type, mesh=mesh, scratch_types=scratch,
        compiler_params=pltpu.CompilerParams(use_tc_tiling_on_sc=False))


def _tc_selfproj(x, w_self, b):
    def body(x_ref, ws_ref, b_ref, xs_ref):
        xs_ref[...] = jnp.dot(
            x_ref[...], ws_ref[...],
            preferred_element_type=jnp.float32) + b_ref[...]

    nblk = 10
    rb = N_NODES // nblk
    return pl.pallas_call(
        body,
        grid=(nblk,),
        in_specs=[
            pl.BlockSpec((rb, D), lambda i: (i, 0)),
            pl.BlockSpec((D, D), lambda i: (0, 0)),
            pl.BlockSpec((1, D), lambda i: (0, 0)),
        ],
        out_specs=pl.BlockSpec((rb, D), lambda i: (i, 0)),
        out_shape=jax.ShapeDtypeStruct((N_NODES, D), jnp.float32),
    )(x, w_self, b.reshape(1, D))


def _tc_combine(xs, parts, degp, w_neigh, gamma, beta, relu):
    nblk = 10
    rb = N_NODES // nblk

    def body(xs_ref, p_ref, dg_ref, wn_ref, g_ref, bt_ref, out_ref,
             hbuf, stats, dcol):
        ph = pl.program_id(0)
        i = pl.program_id(1)

        @pl.when(jnp.logical_and(ph == 0, i == 0))
        def _():
            deg = dg_ref[0] + dg_ref[1]
            dcol[...] = (1.0 / jnp.maximum(deg, 1.0))[:, None]

        @pl.when(ph == 0)
        def _():
            p = p_ref[0] + p_ref[1]
            hn = p * dcol[pl.ds(i * rb, rb), :]
            h = xs_ref[...] + jnp.dot(hn, wn_ref[...],
                                      preferred_element_type=jnp.float32)
            hbuf[pl.ds(i * rb, rb), :] = h

            @pl.when(i == 0)
            def _():
                stats[...] = jnp.zeros((2, D), jnp.float32)
            stats[0:1, :] += jnp.sum(h, axis=0, keepdims=True)
            stats[1:2, :] += jnp.sum(h * h, axis=0, keepdims=True)

        @pl.when(ph == 1)
        def _():
            mu = stats[0:1, :] * (1.0 / N_NODES)
            var = stats[1:2, :] * (1.0 / N_NODES) - mu * mu
            h = hbuf[pl.ds(i * rb, rb), :]
            out = (g_ref[...] * (h - mu) * lax.rsqrt(var + BN_EPS)
                   + bt_ref[...])
            if relu:
                out = jnp.maximum(out, 0.0)
            out_ref[...] = out

    return pl.pallas_call(
        body,
        grid=(2, nblk),
        in_specs=[
            pl.BlockSpec((rb, D), lambda ph, i: (i * (1 - ph), 0)),
            pl.BlockSpec((NC, rb, D), lambda ph, i: (0, i * (1 - ph), 0)),
            pl.BlockSpec((NC, NPAD), lambda ph, i: (0, 0)),
            pl.BlockSpec((D, D), lambda ph, i: (0, 0)),
            pl.BlockSpec((1, D), lambda ph, i: (0, 0)),
            pl.BlockSpec((1, D), lambda ph, i: (0, 0)),
        ],
        out_specs=pl.BlockSpec((rb, D), lambda ph, i: (i * ph, 0)),
        out_shape=jax.ShapeDtypeStruct((N_NODES, D), jnp.float32),
        scratch_shapes=[
            pltpu.VMEM((N_NODES, D), jnp.float32),
            pltpu.VMEM((2, D), jnp.float32),
            pltpu.VMEM((NPAD, 1), jnp.float32),
        ],
    )(xs, parts, degp, w_neigh, gamma.reshape(1, D), beta.reshape(1, D))


def kernel(features, edge_index, W_self1, W_neigh1, b1, gamma1, beta1,
           W_self2, W_neigh2, b2, gamma2, beta2):
    ei = edge_index.astype(jnp.int32)

    parts1, degp = _sc_aggregate(with_deg=True)(features, ei)
    xs1 = _tc_selfproj(features, W_self1, b1)
    h1 = _tc_combine(xs1, parts1, degp, W_neigh1, gamma1, beta1, relu=True)
    (parts2,) = _sc_aggregate(with_deg=False)(h1, ei)
    xs2 = _tc_selfproj(h1, W_self2, b2)
    return _tc_combine(xs2, parts2, degp, W_neigh2, gamma2, beta2,
                       relu=False)


# speedup vs baseline: 1.0480x; 1.0480x over previous
import functools

import jax
import jax.numpy as jnp
from jax import lax
from jax.experimental import pallas as pl
from jax.experimental.pallas import tpu as pltpu
from jax.experimental.pallas import tpu_sc as plsc

N_NODES = 10000
N_EDGES = 320000
D = 128
BN_EPS = 1e-5

NC = 2
NS = 16
NW = NC * NS
CHUNK = 80
CPW = N_EDGES // (NW * CHUNK)
ROWS_PER_TILE = 640
NPAD = NS * ROWS_PER_TILE


def _sc_aggregate(with_deg):
    mesh = plsc.VectorSubcoreMesh(
        core_axis_name="c", subcore_axis_name="s",
        num_cores=NC, num_subcores=NS)

    out_type = [jax.ShapeDtypeStruct((NC, NPAD, D), jnp.float32)]
    scratch = [
        pltpu.VMEM_SHARED((NPAD, D), jnp.float32),
        pltpu.VMEM((CPW * CHUNK,), jnp.int32),
        pltpu.VMEM((CHUNK, D), jnp.float32),
        pltpu.VMEM((CHUNK, D), jnp.float32),
        pltpu.VMEM((CHUNK, D), jnp.float32),
        pltpu.VMEM((CHUNK,), jnp.int32),
        pltpu.VMEM((CHUNK,), jnp.int32),
        pltpu.VMEM((CHUNK,), jnp.int32),
        pltpu.SemaphoreType.DMA,
        pltpu.SemaphoreType.DMA,
        pltpu.SemaphoreType.DMA,
        pltpu.SemaphoreType.DMA,
        pltpu.SemaphoreType.DMA,
        pltpu.SemaphoreType.DMA,
        pltpu.SemaphoreType.DMA,
        pltpu.SemaphoreType.DMA,
        pltpu.SemaphoreType.DMA,
    ]
    if with_deg:
        out_type.append(jax.ShapeDtypeStruct((NC, NPAD), jnp.float32))
        scratch += [
            pltpu.VMEM_SHARED((NPAD,), jnp.float32),
            pltpu.VMEM((CHUNK,), jnp.float32),
            pltpu.VMEM((ROWS_PER_TILE,), jnp.float32),
            pltpu.SemaphoreType.DMA,
            pltpu.SemaphoreType.DMA,
            pltpu.SemaphoreType.DMA,
        ]

    def body(xw_hbm, ei_hbm, parts_hbm, *rest):
        if with_deg:
            (degp_hbm, acc, srcv, r0, r1, r2, d0, d1, d2,
             g0, g1, g2, s0, s1, s2, i0, i1, i2,
             dacc, ones, dz, e0, e1, e2) = rest
            semd = (e0, e1, e2)
        else:
            (acc, srcv, r0, r1, r2, d0, d1, d2,
             g0, g1, g2, s0, s1, s2, i0, i1, i2) = rest
        rows = (r0, r1, r2)
        dsti = (d0, d1, d2)
        semg = (g0, g1, g2)
        sems = (s0, s1, s2)
        semi = (i0, i1, i2)

        c = lax.axis_index("c")
        s = lax.axis_index("s")
        wid = s * NC + c
        row0 = s * ROWS_PER_TILE
        ebase = wid * CPW * CHUNK

        pltpu.async_copy(ei_hbm.at[0, pl.ds(ebase, CPW * CHUNK)], srcv,
                         semi[2])

        z16 = jnp.zeros((16,), jnp.float32)

        def zrow(i, carry):
            for j in range(D // 16):
                r0[i, pl.ds(j * 16, 16)] = z16
            return carry
        lax.fori_loop(0, CHUNK, zrow, 0)
        nz = ROWS_PER_TILE // CHUNK
        for k in range(nz):
            pltpu.async_copy(r0, acc.at[pl.ds(row0 + k * CHUNK, CHUNK)],
                             sems[k % 3])

        if with_deg:
            o16 = jnp.ones((16,), jnp.float32)

            def zdeg(i, carry):
                dz[pl.ds(i * 16, 16)] = z16
                return carry
            lax.fori_loop(0, ROWS_PER_TILE // 16, zdeg, 0)
            pltpu.async_copy(dz, dacc.at[pl.ds(row0, ROWS_PER_TILE)],
                             semd[0])

            def fones(i, carry):
                ones[pl.ds(i * 16, 16)] = o16
                return carry
            lax.fori_loop(0, CHUNK // 16, fones, 0)
            pltpu.make_async_copy(dz, dacc.at[pl.ds(row0, ROWS_PER_TILE)],
                                  semd[0]).wait()

        for k in range(nz):
            pltpu.make_async_copy(r0, acc.at[pl.ds(row0 + k * CHUNK, CHUNK)],
                                  sems[k % 3]).wait()
        pltpu.make_async_copy(ei_hbm.at[0, pl.ds(ebase, CPW * CHUNK)], srcv,
                              semi[2]).wait()

        plsc.subcore_barrier()

        def fetch(j, b):
            pltpu.async_copy(ei_hbm.at[1, pl.ds(ebase + j * CHUNK, CHUNK)],
                             dsti[b], semi[b])
            pltpu.async_copy(xw_hbm.at[srcv.at[pl.ds(j * CHUNK, CHUNK)]],
                             rows[b], semg[b])

        def wait_fetch(j, b):
            pltpu.make_async_copy(
                ei_hbm.at[1, pl.ds(ebase + j * CHUNK, CHUNK)],
                dsti[b], semi[b]).wait()
            pltpu.make_async_copy(xw_hbm.at[srcv.at[pl.ds(j * CHUNK, CHUNK)]],
                                  rows[b], semg[b]).wait()

        def scatter(b):
            pltpu.async_copy(rows[b], acc.at[dsti[b]], sems[b], add=True)
            if with_deg:
                pltpu.async_copy(ones, dacc.at[dsti[b]], semd[b], add=True)

        def wait_scatter(b):
            pltpu.make_async_copy(rows[b], acc.at[dsti[b]], sems[b]).wait()
            if with_deg:
                pltpu.make_async_copy(ones, dacc.at[dsti[b]],
                                      semd[b]).wait()

        fetch(0, 0)
        fetch(1, 1)

        def step(i, carry):
            j = 3 * i
            wait_fetch(j, 0)
            scatter(0)

            @pl.when(i > 0)
            def _():
                wait_scatter(2)
            fetch(j + 2, 2)

            wait_fetch(j + 1, 1)
            scatter(1)
            wait_scatter(0)
            fetch(j + 3, 0)

            wait_fetch(j + 2, 2)
            scatter(2)
            wait_scatter(1)
            fetch(j + 4, 1)
            return carry
        lax.fori_loop(0, (CPW - 2) // 3, step, 0)

        wait_fetch(CPW - 2, 0)
        scatter(0)
        wait_fetch(CPW - 1, 1)
        scatter(1)
        wait_scatter(2)
        wait_scatter(0)
        wait_scatter(1)

        plsc.subcore_barrier()

        pltpu.async_copy(acc.at[pl.ds(row0, ROWS_PER_TILE)],
                         parts_hbm.at[c, pl.ds(row0, ROWS_PER_TILE)],
                         sems[0])
        if with_deg:
            pltpu.async_copy(dacc.at[pl.ds(row0, ROWS_PER_TILE)],
                             degp_hbm.at[c, pl.ds(row0, ROWS_PER_TILE)],
                             semd[0])
            pltpu.make_async_copy(
                dacc.at[pl.ds(row0, ROWS_PER_TILE)],
                degp_hbm.at[c, pl.ds(row0, ROWS_PER_TILE)], semd[0]).wait()
        pltpu.make_async_copy(
            acc.at[pl.ds(row0, ROWS_PER_TILE)],
            parts_hbm.at[c, pl.ds(row0, ROWS_PER_TILE)], sems[0]).wait()

    return pl.kernel(
        body, out_type=out_type, mesh=mesh, scratch_types=scratch,
        compiler_params=pltpu.CompilerParams(use_tc_tiling_on_sc=False))


def _tc_selfproj(x, w_self, b):
    def body(x_ref, ws_ref, b_ref, xs_ref):
        xs_ref[...] = jnp.dot(
            x_ref[...], ws_ref[...],
            preferred_element_type=jnp.float32) + b_ref[...]

    nblk = 10
    rb = N_NODES // nblk
    return pl.pallas_call(
        body,
        grid=(nblk,),
        in_specs=[
            pl.BlockSpec((rb, D), lambda i: (i, 0)),
            pl.BlockSpec((D, D), lambda i: (0, 0)),
            pl.BlockSpec((1, D), lambda i: (0, 0)),
        ],
        out_specs=pl.BlockSpec((rb, D), lambda i: (i, 0)),
        out_shape=jax.ShapeDtypeStruct((N_NODES, D), jnp.float32),
    )(x, w_self, b.reshape(1, D))


def _tc_combine(xs, parts, degp, w_neigh, gamma, beta, relu):
    def body(xs_ref, p_ref, dg_ref, wn_ref, g_ref, bt_ref, out_ref):
        p = p_ref[0, :N_NODES, :] + p_ref[1, :N_NODES, :]
        deg = dg_ref[0, :N_NODES] + dg_ref[1, :N_NODES]
        hn = p / jnp.maximum(deg, 1.0)[:, None]
        h = xs_ref[...] + jnp.dot(hn, wn_ref[...],
                                  preferred_element_type=jnp.float32)
        mu = jnp.mean(h, axis=0, keepdims=True)
        var = jnp.mean((h - mu) ** 2, axis=0, keepdims=True)
        out = g_ref[...] * (h - mu) * lax.rsqrt(var + BN_EPS) + bt_ref[...]
        if relu:
            out = jnp.maximum(out, 0.0)
        out_ref[...] = out

    return pl.pallas_call(
        body,
        out_shape=jax.ShapeDtypeStruct((N_NODES, D), jnp.float32),
    )(xs, parts, degp, w_neigh, gamma.reshape(1, D), beta.reshape(1, D))


def kernel(features, edge_index, W_self1, W_neigh1, b1, gamma1, beta1,
           W_self2, W_neigh2, b2, gamma2, beta2):
    ei = edge_index.astype(jnp.int32)

    parts1, degp = _sc_aggregate(with_deg=True)(features, ei)
    xs1 = _tc_selfproj(features, W_self1, b1)
    h1 = _tc_combine(xs1, parts1, degp, W_neigh1, gamma1, beta1, relu=True)
    (parts2,) = _sc_aggregate(with_deg=False)(h1, ei)
    xs2 = _tc_selfproj(h1, W_self2, b2)
    return _tc_combine(xs2, parts2, degp, W_neigh2, gamma2, beta2,
                       relu=False)
